# baseline (speedup 1.0000x reference)
import functools

import jax
import jax.numpy as jnp
from jax import lax
from jax.experimental import pallas as pl
from jax.experimental.pallas import tpu as pltpu
from jax.experimental.pallas import tpu_sc as plsc

_N = 10000
_E = 160000
_IN = 16
_HID = 16
_OUT = 8
_DE = 4
_MLP_H = 25

_NC = 2
_NS = 16
_NW = _NC * _NS
_CH = 128
_NCHUNK = 40
_EPT = _CH * _NCHUNK
_E_PAD = _EPT * _NW
_N_SC = 10240
_RPT = _N_SC // _NS

@functools.cache
def _mesh():
  return plsc.VectorSubcoreMesh(core_axis_name="c", subcore_axis_name="s")


_SC_PARAMS = pltpu.CompilerParams(use_tc_tiling_on_sc=False)


def _sc_gather(table, idx):
  @functools.partial(
      pl.kernel,
      out_type=jax.ShapeDtypeStruct((_E_PAD, 16), jnp.float32),
      mesh=_mesh(),
      compiler_params=_SC_PARAMS,
      scratch_types=[
          pltpu.VMEM((_NCHUNK, _CH), jnp.int32),
          pltpu.VMEM((_EPT, 16), jnp.float32),
      ],
  )
  def k(table_hbm, idx_hbm, out_hbm, idx_v, rows_v):
    wid = lax.axis_index("c") * _NS + lax.axis_index("s")
    pltpu.sync_copy(idx_hbm.at[wid], idx_v)

    @pl.loop(0, _NCHUNK)
    def _(j):
      pltpu.sync_copy(table_hbm.at[idx_v.at[j]],
                      rows_v.at[pl.ds(j * _CH, _CH)])

    pltpu.sync_copy(rows_v, out_hbm.at[pl.ds(wid * _EPT, _EPT)])

  return k(table, idx)


def _sc_scatter_add(msg, idx):
  @functools.partial(
      pl.kernel,
      out_type=jax.ShapeDtypeStruct((_NC, _N_SC, 16), jnp.float32),
      mesh=_mesh(),
      compiler_params=_SC_PARAMS,
      scratch_types=[
          pltpu.VMEM((_NCHUNK, _CH), jnp.int32),
          pltpu.VMEM((_CH, 16), jnp.float32),
          pltpu.VMEM((_RPT, 16), jnp.float32),
          pltpu.VMEM_SHARED((_N_SC, 16), jnp.float32),
      ],
  )
  def k(msg_hbm, idx_hbm, out_hbm, idx_v, msg_v, stage_v, acc_sh):
    c = lax.axis_index("c")
    s = lax.axis_index("s")
    wid = c * _NS + s
    pltpu.sync_copy(idx_hbm.at[wid], idx_v)

    @pl.loop(0, _RPT)
    def _(r):
      stage_v[r, :] = jnp.zeros((16,), jnp.float32)

    pltpu.sync_copy(stage_v, acc_sh.at[pl.ds(s * _RPT, _RPT)])
    plsc.subcore_barrier()

    base = wid * _EPT

    @pl.loop(0, _NCHUNK)
    def _(j):
      pltpu.sync_copy(msg_hbm.at[pl.ds(base + j * _CH, _CH)], msg_v)
      pltpu.sync_copy(msg_v, acc_sh.at[idx_v.at[j]], add=True)

    plsc.subcore_barrier()

    pltpu.sync_copy(acc_sh.at[pl.ds(s * _RPT, _RPT)], stage_v)
    pltpu.sync_copy(stage_v, out_hbm.at[c, pl.ds(s * _RPT, _RPT)])

  return k(msg, idx)


def _sc_count(idx):
  @functools.partial(
      pl.kernel,
      out_type=jax.ShapeDtypeStruct((_NC, _N_SC), jnp.float32),
      mesh=_mesh(),
      compiler_params=_SC_PARAMS,
      scratch_types=[
          pltpu.VMEM((_NCHUNK, _CH), jnp.int32),
          pltpu.VMEM((_CH,), jnp.float32),
          pltpu.VMEM((_RPT,), jnp.float32),
          pltpu.VMEM_SHARED((_N_SC,), jnp.float32),
      ],
  )
  def k(idx_hbm, out_hbm, idx_v, ones_v, stage_v, acc_sh):
    c = lax.axis_index("c")
    s = lax.axis_index("s")
    wid = c * _NS + s
    pltpu.sync_copy(idx_hbm.at[wid], idx_v)

    @pl.loop(0, _CH // 16)
    def _(r):
      ones_v[pl.ds(r * 16, 16)] = jnp.ones((16,), jnp.float32)

    @pl.loop(0, _RPT // 16)
    def _(r):
      stage_v[pl.ds(r * 16, 16)] = jnp.zeros((16,), jnp.float32)

    pltpu.sync_copy(stage_v, acc_sh.at[pl.ds(s * _RPT, _RPT)])
    plsc.subcore_barrier()

    @pl.loop(0, _NCHUNK)
    def _(j):
      pltpu.sync_copy(ones_v, acc_sh.at[idx_v.at[j]], add=True)

    plsc.subcore_barrier()

    pltpu.sync_copy(acc_sh.at[pl.ds(s * _RPT, _RPT)], stage_v)
    pltpu.sync_copy(stage_v, out_hbm.at[c, pl.ds(s * _RPT, _RPT)])

  return k(idx)


_EBLK = 2048


def _dense_body(ea_ref, Wa_ref, ba_ref, Wb_ref, bb_ref, xs_ref, R_ref, T_ref,
                msg_ref):
  dot = functools.partial(jnp.dot, preferred_element_type=jnp.float32,
                          precision=lax.Precision.DEFAULT)
  h = jnp.maximum(dot(ea_ref[...], Wa_ref[...]) + ba_ref[...], 0.0)
  w = dot(h, Wb_ref[...]) + bb_ref[...]
  xrep = dot(xs_ref[...], R_ref[...])
  msg_ref[...] = dot(xrep * w, T_ref[...])


def _tc_dense(ea, Wa, ba, Wb, bb, xs, R, T):
  grid = _E_PAD // _EBLK
  wdim = Wb.shape[1]
  full = lambda *shape: pl.BlockSpec(shape, lambda i: (0,) * len(shape))
  return pl.pallas_call(
      _dense_body,
      grid=(grid,),
      in_specs=[
          pl.BlockSpec((_EBLK, _DE), lambda i: (i, 0)),
          full(_DE, _MLP_H),
          full(1, _MLP_H),
          full(_MLP_H, wdim),
          full(1, wdim),
          pl.BlockSpec((_EBLK, 16), lambda i: (i, 0)),
          full(16, wdim),
          full(wdim, 16),
      ],
      out_specs=pl.BlockSpec((_EBLK, 16), lambda i: (i, 0)),
      out_shape=jax.ShapeDtypeStruct((_E_PAD, 16), jnp.float32),
  )(ea, Wa, ba, Wb, bb, xs, R, T)


def _combine_body(relu, s0_ref, s1_ref, c0_ref, c1_ref, x_ref, root_ref,
                  bias_ref, o_ref):
  cnt = jnp.maximum(c0_ref[...] + c1_ref[...], 1.0)
  out = (s0_ref[...] + s1_ref[...]) / cnt
  out = out + jnp.dot(x_ref[...], root_ref[...],
                      preferred_element_type=jnp.float32) + bias_ref[...]
  if relu:
    out = jnp.maximum(out, 0.0)
  o_ref[...] = out


def _tc_combine(s0, s1, c0, c1, x, root, bias, relu):
  return pl.pallas_call(
      functools.partial(_combine_body, relu),
      out_shape=jax.ShapeDtypeStruct((_N_SC, 16), jnp.float32),
  )(s0, s1, c0, c1, x, root, bias)


def kernel(x, edge_index, edge_attr, W1a, b1a, W1b, b1b, root1, bias1,
           W2a, b2a, W2b, b2b, root2, bias2):
  src = edge_index[0]
  dst = edge_index[1]
  pad = _E_PAD - _E
  src_p = jnp.concatenate([src, jnp.zeros((pad,), jnp.int32)])
  src_p = src_p.reshape(_NW, _NCHUNK, _CH)
  dst_p = jnp.concatenate([dst, jnp.full((pad,), _N, jnp.int32)])
  dst_p = dst_p.reshape(_NW, _NCHUNK, _CH)
  ea_p = jnp.pad(edge_attr, ((0, pad), (0, 0)))
  x_p = jnp.pad(x, ((0, _N_SC - _N), (0, 0)))

  R16 = jnp.kron(jnp.eye(16, dtype=jnp.float32),
                 jnp.ones((1, 16), jnp.float32))
  T16 = jnp.kron(jnp.ones((16, 1), jnp.float32),
                 jnp.eye(16, dtype=jnp.float32))

  W2b_p = jnp.pad(W2b.reshape(_MLP_H, _HID, _OUT),
                  ((0, 0), (0, 0), (0, 16 - _OUT))).reshape(_MLP_H, _HID * 16)
  b2b_p = jnp.pad(b2b.reshape(_HID, _OUT),
                  ((0, 0), (0, 16 - _OUT))).reshape(1, _HID * 16)
  root2_p = jnp.pad(root2, ((0, 0), (0, 16 - _OUT)))
  bias2_p = jnp.pad(bias2, ((0, 16 - _OUT))).reshape(1, 16)

  b1a_ = b1a.reshape(1, _MLP_H)
  b1b_ = b1b.reshape(1, _IN * _HID)
  b2a_ = b2a.reshape(1, _MLP_H)
  bias1_ = bias1.reshape(1, _HID)

  cnt = _sc_count(dst_p)
  c0 = cnt[0].reshape(_N_SC, 1)
  c1 = cnt[1].reshape(_N_SC, 1)

  xs1 = _sc_gather(x, src_p)
  msg1 = _tc_dense(ea_p, W1a, b1a_, W1b, b1b_, xs1, R16, T16)
  s1 = _sc_scatter_add(msg1, dst_p)
  h = _tc_combine(s1[0], s1[1], c0, c1, x_p, root1, bias1_, relu=True)

  xs2 = _sc_gather(h, src_p)
  msg2 = _tc_dense(ea_p, W2a, b2a_, W2b_p, b2b_p, xs2, R16, T16)
  s2 = _sc_scatter_add(msg2, dst_p)
  out = _tc_combine(s2[0], s2[1], c0, c1, h, root2_p, bias2_p, relu=False)

  return out[:_N, :_OUT]


# speedup vs baseline: 1.0518x; 1.0518x over previous
import functools

import jax
import jax.numpy as jnp
from jax import lax
from jax.experimental import pallas as pl
from jax.experimental.pallas import tpu as pltpu
from jax.experimental.pallas import tpu_sc as plsc

_N = 10000
_E = 160000
_IN = 16
_HID = 16
_OUT = 8
_DE = 4
_MLP_H = 25

_NC = 2
_NS = 16
_NW = _NC * _NS
_CH = 128
_NCHUNK = 40
_EPT = _CH * _NCHUNK
_E_PAD = _EPT * _NW
_N_SC = 10240
_RPT = _N_SC // _NS


@functools.cache
def _mesh():
  return plsc.VectorSubcoreMesh(core_axis_name="c", subcore_axis_name="s")


_SC_PARAMS = pltpu.CompilerParams(use_tc_tiling_on_sc=False)


def _sc_gather_count(table, src_idx, dst_idx):
  @functools.partial(
      pl.kernel,
      out_type=[
          jax.ShapeDtypeStruct((_E_PAD, 16), jnp.float32),
          jax.ShapeDtypeStruct((_NC, _N_SC), jnp.float32),
      ],
      mesh=_mesh(),
      compiler_params=_SC_PARAMS,
      scratch_types=[
          pltpu.VMEM((_NCHUNK, _CH), jnp.int32),
          pltpu.VMEM((_NCHUNK, _CH), jnp.int32),
          pltpu.VMEM((_EPT, 16), jnp.float32),
          pltpu.VMEM((_CH,), jnp.float32),
          pltpu.VMEM((_RPT,), jnp.float32),
          pltpu.VMEM_SHARED((_N_SC,), jnp.float32),
          pltpu.SemaphoreType.DMA,
      ],
  )
  def k(table_hbm, sidx_hbm, didx_hbm, xs_hbm, cnt_hbm,
        sidx_v, didx_v, rows_v, ones_v, cstage_v, cnt_sh, gsem):
    c = lax.axis_index("c")
    s = lax.axis_index("s")
    wid = c * _NS + s
    pltpu.sync_copy(sidx_hbm.at[wid], sidx_v)
    pltpu.sync_copy(didx_hbm.at[wid], didx_v)

    @pl.loop(0, _NCHUNK)
    def _(j):
      pltpu.async_copy(table_hbm.at[sidx_v.at[j]],
                       rows_v.at[pl.ds(j * _CH, _CH)], gsem)

    @pl.loop(0, _CH // 16)
    def _(r):
      ones_v[pl.ds(r * 16, 16)] = jnp.ones((16,), jnp.float32)

    @pl.loop(0, _RPT // 16)
    def _(r):
      cstage_v[pl.ds(r * 16, 16)] = jnp.zeros((16,), jnp.float32)

    pltpu.sync_copy(cstage_v, cnt_sh.at[pl.ds(s * _RPT, _RPT)])
    plsc.subcore_barrier()

    @pl.loop(0, _NCHUNK)
    def _(j):
      pltpu.sync_copy(ones_v, cnt_sh.at[didx_v.at[j]], add=True)

    plsc.subcore_barrier()

    pltpu.sync_copy(cnt_sh.at[pl.ds(s * _RPT, _RPT)], cstage_v)
    pltpu.sync_copy(cstage_v, cnt_hbm.at[c, pl.ds(s * _RPT, _RPT)])

    @pl.loop(0, _NCHUNK)
    def _(j):
      pltpu.make_async_copy(table_hbm.at[sidx_v.at[j]],
                            rows_v.at[pl.ds(j * _CH, _CH)], gsem).wait()

    pltpu.sync_copy(rows_v, xs_hbm.at[pl.ds(wid * _EPT, _EPT)])

  return k(table, src_idx, dst_idx)


def _sc_gather(table, idx):
  @functools.partial(
      pl.kernel,
      out_type=jax.ShapeDtypeStruct((_E_PAD, 16), jnp.float32),
      mesh=_mesh(),
      compiler_params=_SC_PARAMS,
      scratch_types=[
          pltpu.VMEM((_NCHUNK, _CH), jnp.int32),
          pltpu.VMEM((_EPT, 16), jnp.float32),
          pltpu.SemaphoreType.DMA,
      ],
  )
  def k(table_hbm, idx_hbm, out_hbm, idx_v, rows_v, gsem):
    wid = lax.axis_index("c") * _NS + lax.axis_index("s")
    pltpu.sync_copy(idx_hbm.at[wid], idx_v)

    @pl.loop(0, _NCHUNK)
    def _(j):
      pltpu.async_copy(table_hbm.at[idx_v.at[j]],
                       rows_v.at[pl.ds(j * _CH, _CH)], gsem)

    @pl.loop(0, _NCHUNK)
    def _(j):
      pltpu.make_async_copy(table_hbm.at[idx_v.at[j]],
                            rows_v.at[pl.ds(j * _CH, _CH)], gsem).wait()

    pltpu.sync_copy(rows_v, out_hbm.at[pl.ds(wid * _EPT, _EPT)])

  return k(table, idx)


def _sc_scatter_add(msg, idx):
  @functools.partial(
      pl.kernel,
      out_type=jax.ShapeDtypeStruct((_NC, _N_SC, 16), jnp.float32),
      mesh=_mesh(),
      compiler_params=_SC_PARAMS,
      scratch_types=[
          pltpu.VMEM((_NCHUNK, _CH), jnp.int32),
          pltpu.VMEM((_CH, 16), jnp.float32),
          pltpu.VMEM((_CH, 16), jnp.float32),
          pltpu.VMEM((_RPT, 16), jnp.float32),
          pltpu.VMEM_SHARED((_N_SC, 16), jnp.float32),
          pltpu.SemaphoreType.DMA,
          pltpu.SemaphoreType.DMA,
      ],
  )
  def k(msg_hbm, idx_hbm, out_hbm, idx_v, msg_a, msg_b, stage_v, acc_sh,
        sem_a, sem_b):
    c = lax.axis_index("c")
    s = lax.axis_index("s")
    wid = c * _NS + s
    pltpu.sync_copy(idx_hbm.at[wid], idx_v)

    @pl.loop(0, _RPT)
    def _(r):
      stage_v[r, :] = jnp.zeros((16,), jnp.float32)

    pltpu.sync_copy(stage_v, acc_sh.at[pl.ds(s * _RPT, _RPT)])
    plsc.subcore_barrier()

    base = wid * _EPT

    @pl.loop(0, _NCHUNK // 2)
    def _(p):
      j0 = 2 * p
      j1 = 2 * p + 1
      a = pltpu.async_copy(msg_hbm.at[pl.ds(base + j0 * _CH, _CH)], msg_a,
                           sem_a)
      b = pltpu.async_copy(msg_hbm.at[pl.ds(base + j1 * _CH, _CH)], msg_b,
                           sem_b)
      a.wait()
      pltpu.sync_copy(msg_a, acc_sh.at[idx_v.at[j0]], add=True)
      b.wait()
      pltpu.sync_copy(msg_b, acc_sh.at[idx_v.at[j1]], add=True)

    plsc.subcore_barrier()

    pltpu.sync_copy(acc_sh.at[pl.ds(s * _RPT, _RPT)], stage_v)
    pltpu.sync_copy(stage_v, out_hbm.at[c, pl.ds(s * _RPT, _RPT)])

  return k(msg, idx)


_EBLK = 2048


def _dense_body(ea_ref, Wa_ref, ba_ref, Wb_ref, bb_ref, xs_ref, R_ref, T_ref,
                msg_ref):
  dot = functools.partial(jnp.dot, preferred_element_type=jnp.float32)
  h = jnp.maximum(dot(ea_ref[...], Wa_ref[...]) + ba_ref[...], 0.0)
  w = dot(h, Wb_ref[...]) + bb_ref[...]
  xrep = dot(xs_ref[...], R_ref[...])
  msg_ref[...] = dot(xrep * w, T_ref[...])


def _tc_dense(ea, Wa, ba, Wb, bb, xs, R, T):
  grid = _E_PAD // _EBLK
  wdim = Wb.shape[1]
  full = lambda *shape: pl.BlockSpec(shape, lambda i: (0,) * len(shape))
  return pl.pallas_call(
      _dense_body,
      grid=(grid,),
      in_specs=[
          pl.BlockSpec((_EBLK, _DE), lambda i: (i, 0)),
          full(_DE, _MLP_H),
          full(1, _MLP_H),
          full(_MLP_H, wdim),
          full(1, wdim),
          pl.BlockSpec((_EBLK, 16), lambda i: (i, 0)),
          full(16, wdim),
          full(wdim, 16),
      ],
      out_specs=pl.BlockSpec((_EBLK, 16), lambda i: (i, 0)),
      out_shape=jax.ShapeDtypeStruct((_E_PAD, 16), jnp.float32),
  )(ea, Wa, ba, Wb, bb, xs, R, T)


def _combine_body(relu, s0_ref, s1_ref, c0_ref, c1_ref, x_ref, root_ref,
                  bias_ref, o_ref):
  cnt = jnp.maximum(c0_ref[...] + c1_ref[...], 1.0)
  out = (s0_ref[...] + s1_ref[...]) / cnt
  out = out + jnp.dot(x_ref[...], root_ref[...],
                      preferred_element_type=jnp.float32) + bias_ref[...]
  if relu:
    out = jnp.maximum(out, 0.0)
  o_ref[...] = out


def _tc_combine(s0, s1, c0, c1, x, root, bias, relu):
  return pl.pallas_call(
      functools.partial(_combine_body, relu),
      out_shape=jax.ShapeDtypeStruct((_N_SC, 16), jnp.float32),
  )(s0, s1, c0, c1, x, root, bias)


def kernel(x, edge_index, edge_attr, W1a, b1a, W1b, b1b, root1, bias1,
           W2a, b2a, W2b, b2b, root2, bias2):
  src = edge_index[0]
  dst = edge_index[1]
  pad = _E_PAD - _E
  src_p = jnp.concatenate([src, jnp.zeros((pad,), jnp.int32)])
  src_p = src_p.reshape(_NW, _NCHUNK, _CH)
  dst_p = jnp.concatenate([dst, jnp.full((pad,), _N, jnp.int32)])
  dst_p = dst_p.reshape(_NW, _NCHUNK, _CH)
  ea_p = jnp.pad(edge_attr, ((0, pad), (0, 0)))
  x_p = jnp.pad(x, ((0, _N_SC - _N), (0, 0)))

  R16 = jnp.kron(jnp.eye(16, dtype=jnp.float32),
                 jnp.ones((1, 16), jnp.float32))
  T16 = jnp.kron(jnp.ones((16, 1), jnp.float32),
                 jnp.eye(16, dtype=jnp.float32))

  W2b_p = jnp.pad(W2b.reshape(_MLP_H, _HID, _OUT),
                  ((0, 0), (0, 0), (0, 16 - _OUT))).reshape(_MLP_H, _HID * 16)
  b2b_p = jnp.pad(b2b.reshape(_HID, _OUT),
                  ((0, 0), (0, 16 - _OUT))).reshape(1, _HID * 16)
  root2_p = jnp.pad(root2, ((0, 0), (0, 16 - _OUT)))
  bias2_p = jnp.pad(bias2, ((0, 16 - _OUT))).reshape(1, 16)

  b1a_ = b1a.reshape(1, _MLP_H)
  b1b_ = b1b.reshape(1, _IN * _HID)
  b2a_ = b2a.reshape(1, _MLP_H)
  bias1_ = bias1.reshape(1, _HID)

  xs1, cnt = _sc_gather_count(x, src_p, dst_p)
  c0 = cnt[0].reshape(_N_SC, 1)
  c1 = cnt[1].reshape(_N_SC, 1)

  msg1 = _tc_dense(ea_p, W1a, b1a_, W1b, b1b_, xs1, R16, T16)
  s1 = _sc_scatter_add(msg1, dst_p)
  h = _tc_combine(s1[0], s1[1], c0, c1, x_p, root1, bias1_, relu=True)

  xs2 = _sc_gather(h, src_p)
  msg2 = _tc_dense(ea_p, W2a, b2a_, W2b_p, b2b_p, xs2, R16, T16)
  s2 = _sc_scatter_add(msg2, dst_p)
  out = _tc_combine(s2[0], s2[1], c0, c1, h, root2_p, bias2_p, relu=False)

  return out[:_N, :_OUT]


# speedup vs baseline: 1.2688x; 1.2063x over previous
import functools

import jax
import jax.numpy as jnp
from jax import lax
from jax.experimental import pallas as pl
from jax.experimental.pallas import tpu as pltpu
from jax.experimental.pallas import tpu_sc as plsc

_N = 10000
_E = 160000
_IN = 16
_HID = 16
_OUT = 8
_DE = 4
_MLP_H = 25

_NC = 2
_NS = 16
_NW = _NC * _NS
_CH = 128
_NCHUNK = 40
_EPT = _CH * _NCHUNK
_E_PAD = _EPT * _NW
_N_SC = 10240
_RPT = _N_SC // _NS


@functools.cache
def _mesh():
  return plsc.VectorSubcoreMesh(core_axis_name="c", subcore_axis_name="s")


_SC_PARAMS = pltpu.CompilerParams(use_tc_tiling_on_sc=False)


def _sc_gather_count(table, src_idx, dst_idx):
  @functools.partial(
      pl.kernel,
      out_type=[
          jax.ShapeDtypeStruct((_E_PAD, 16), jnp.float32),
          jax.ShapeDtypeStruct((_NC, _N_SC), jnp.float32),
      ],
      mesh=_mesh(),
      compiler_params=_SC_PARAMS,
      scratch_types=[
          pltpu.VMEM((_NCHUNK, _CH), jnp.int32),
          pltpu.VMEM((_NCHUNK, _CH), jnp.int32),
          pltpu.VMEM((_EPT, 16), jnp.float32),
          pltpu.VMEM((_CH,), jnp.float32),
          pltpu.VMEM((_RPT,), jnp.float32),
          pltpu.VMEM_SHARED((_N_SC,), jnp.float32),
          pltpu.SemaphoreType.DMA,
      ],
  )
  def k(table_hbm, sidx_hbm, didx_hbm, xs_hbm, cnt_hbm,
        sidx_v, didx_v, rows_v, ones_v, cstage_v, cnt_sh, gsem):
    c = lax.axis_index("c")
    s = lax.axis_index("s")
    wid = c * _NS + s
    pltpu.sync_copy(sidx_hbm.at[wid], sidx_v)
    pltpu.sync_copy(didx_hbm.at[wid], didx_v)

    @pl.loop(0, _NCHUNK)
    def _(j):
      pltpu.async_copy(table_hbm.at[sidx_v.at[j]],
                       rows_v.at[pl.ds(j * _CH, _CH)], gsem)

    @pl.loop(0, _CH // 16)
    def _(r):
      ones_v[pl.ds(r * 16, 16)] = jnp.ones((16,), jnp.float32)

    @pl.loop(0, _RPT // 16)
    def _(r):
      cstage_v[pl.ds(r * 16, 16)] = jnp.zeros((16,), jnp.float32)

    pltpu.sync_copy(cstage_v, cnt_sh.at[pl.ds(s * _RPT, _RPT)])
    plsc.subcore_barrier()

    @pl.loop(0, _NCHUNK)
    def _(j):
      pltpu.sync_copy(ones_v, cnt_sh.at[didx_v.at[j]], add=True)

    plsc.subcore_barrier()

    pltpu.sync_copy(cnt_sh.at[pl.ds(s * _RPT, _RPT)], cstage_v)
    pltpu.sync_copy(cstage_v, cnt_hbm.at[c, pl.ds(s * _RPT, _RPT)])

    @pl.loop(0, _NCHUNK)
    def _(j):
      pltpu.make_async_copy(table_hbm.at[sidx_v.at[j]],
                            rows_v.at[pl.ds(j * _CH, _CH)], gsem).wait()

    pltpu.sync_copy(rows_v, xs_hbm.at[pl.ds(wid * _EPT, _EPT)])

  return k(table, src_idx, dst_idx)


def _sc_gather(table, idx):
  @functools.partial(
      pl.kernel,
      out_type=jax.ShapeDtypeStruct((_E_PAD, 16), jnp.float32),
      mesh=_mesh(),
      compiler_params=_SC_PARAMS,
      scratch_types=[
          pltpu.VMEM((_NCHUNK, _CH), jnp.int32),
          pltpu.VMEM((_EPT, 16), jnp.float32),
          pltpu.SemaphoreType.DMA,
      ],
  )
  def k(table_hbm, idx_hbm, out_hbm, idx_v, rows_v, gsem):
    wid = lax.axis_index("c") * _NS + lax.axis_index("s")
    pltpu.sync_copy(idx_hbm.at[wid], idx_v)

    @pl.loop(0, _NCHUNK)
    def _(j):
      pltpu.async_copy(table_hbm.at[idx_v.at[j]],
                       rows_v.at[pl.ds(j * _CH, _CH)], gsem)

    @pl.loop(0, _NCHUNK)
    def _(j):
      pltpu.make_async_copy(table_hbm.at[idx_v.at[j]],
                            rows_v.at[pl.ds(j * _CH, _CH)], gsem).wait()

    pltpu.sync_copy(rows_v, out_hbm.at[pl.ds(wid * _EPT, _EPT)])

  return k(table, idx)


def _sc_scatter_add(msg, idx):
  @functools.partial(
      pl.kernel,
      out_type=jax.ShapeDtypeStruct((_NC, _N_SC, 16), jnp.float32),
      mesh=_mesh(),
      compiler_params=_SC_PARAMS,
      scratch_types=[
          pltpu.VMEM((_NCHUNK, _CH), jnp.int32),
          pltpu.VMEM((_CH, 16), jnp.float32),
          pltpu.VMEM((_CH, 16), jnp.float32),
          pltpu.VMEM((_RPT, 16), jnp.float32),
          pltpu.VMEM_SHARED((_N_SC, 16), jnp.float32),
          pltpu.SemaphoreType.DMA,
          pltpu.SemaphoreType.DMA,
      ],
  )
  def k(msg_hbm, idx_hbm, out_hbm, idx_v, msg_a, msg_b, stage_v, acc_sh,
        sem_a, sem_b):
    c = lax.axis_index("c")
    s = lax.axis_index("s")
    wid = c * _NS + s
    pltpu.sync_copy(idx_hbm.at[wid], idx_v)

    @pl.loop(0, _RPT)
    def _(r):
      stage_v[r, :] = jnp.zeros((16,), jnp.float32)

    pltpu.sync_copy(stage_v, acc_sh.at[pl.ds(s * _RPT, _RPT)])
    plsc.subcore_barrier()

    base = wid * _EPT

    @pl.loop(0, _NCHUNK // 2)
    def _(p):
      j0 = 2 * p
      j1 = 2 * p + 1
      a = pltpu.async_copy(msg_hbm.at[pl.ds(base + j0 * _CH, _CH)], msg_a,
                           sem_a)
      b = pltpu.async_copy(msg_hbm.at[pl.ds(base + j1 * _CH, _CH)], msg_b,
                           sem_b)
      a.wait()
      pltpu.sync_copy(msg_a, acc_sh.at[idx_v.at[j0]], add=True)
      b.wait()
      pltpu.sync_copy(msg_b, acc_sh.at[idx_v.at[j1]], add=True)

    plsc.subcore_barrier()

    pltpu.sync_copy(acc_sh.at[pl.ds(s * _RPT, _RPT)], stage_v)
    pltpu.sync_copy(stage_v, out_hbm.at[c, pl.ds(s * _RPT, _RPT)])

  return k(msg, idx)


_EBLK = 8192


def _dense_body(ea_ref, Wa_ref, ba_ref, Wb_ref, bb_ref, xs_ref, R_ref,
                msg_ref):
  dot = functools.partial(jnp.dot, preferred_element_type=jnp.float32)
  bf = jnp.bfloat16
  h = jnp.maximum(dot(ea_ref[...], Wa_ref[...]) + ba_ref[...], 0.0)
  w = dot(h.astype(bf), Wb_ref[...]) + bb_ref[...]
  xrep = dot(xs_ref[...].astype(bf), R_ref[...])
  p = xrep * w
  p = p[:, :128] + p[:, 128:]
  p = p[:, :64] + p[:, 64:]
  p = p[:, :32] + p[:, 32:]
  msg_ref[...] = p[:, :16] + p[:, 16:]


def _tc_dense(ea, Wa, ba, Wb, bb, xs, R):
  grid = _E_PAD // _EBLK
  wdim = Wb.shape[1]
  full = lambda *shape: pl.BlockSpec(shape, lambda i: (0,) * len(shape))
  return pl.pallas_call(
      _dense_body,
      grid=(grid,),
      in_specs=[
          pl.BlockSpec((_EBLK, _DE), lambda i: (i, 0)),
          full(_DE, _MLP_H),
          full(1, _MLP_H),
          full(_MLP_H, wdim),
          full(1, wdim),
          pl.BlockSpec((_EBLK, 16), lambda i: (i, 0)),
          full(16, wdim),
      ],
      out_specs=pl.BlockSpec((_EBLK, 16), lambda i: (i, 0)),
      out_shape=jax.ShapeDtypeStruct((_E_PAD, 16), jnp.float32),
  )(ea, Wa, ba, Wb, bb, xs, R)


def _combine_body(relu, s0_ref, s1_ref, c0_ref, c1_ref, x_ref, root_ref,
                  bias_ref, o_ref):
  cnt = jnp.maximum(c0_ref[...] + c1_ref[...], 1.0)
  out = (s0_ref[...] + s1_ref[...]) / cnt
  out = out + jnp.dot(x_ref[...], root_ref[...],
                      preferred_element_type=jnp.float32) + bias_ref[...]
  if relu:
    out = jnp.maximum(out, 0.0)
  o_ref[...] = out


def _tc_combine(s0, s1, c0, c1, x, root, bias, relu):
  return pl.pallas_call(
      functools.partial(_combine_body, relu),
      out_shape=jax.ShapeDtypeStruct((_N_SC, 16), jnp.float32),
  )(s0, s1, c0, c1, x, root, bias)


def kernel(x, edge_index, edge_attr, W1a, b1a, W1b, b1b, root1, bias1,
           W2a, b2a, W2b, b2b, root2, bias2):
  src = edge_index[0]
  dst = edge_index[1]
  pad = _E_PAD - _E
  src_p = jnp.concatenate([src, jnp.zeros((pad,), jnp.int32)])
  src_p = src_p.reshape(_NW, _NCHUNK, _CH)
  dst_p = jnp.concatenate([dst, jnp.full((pad,), _N, jnp.int32)])
  dst_p = dst_p.reshape(_NW, _NCHUNK, _CH)
  ea_p = jnp.pad(edge_attr, ((0, pad), (0, 0))).astype(jnp.bfloat16)
  x_p = jnp.pad(x, ((0, _N_SC - _N), (0, 0)))

  R16 = jnp.kron(jnp.eye(16, dtype=jnp.bfloat16),
                 jnp.ones((1, 16), jnp.bfloat16))

  W2b_p = jnp.pad(W2b.reshape(_MLP_H, _HID, _OUT),
                  ((0, 0), (0, 0), (0, 16 - _OUT))).reshape(_MLP_H, _HID * 16)
  W1a_b = W1a.astype(jnp.bfloat16)
  W2a_b = W2a.astype(jnp.bfloat16)
  W1b_b = W1b.astype(jnp.bfloat16)
  W2b_b = W2b_p.astype(jnp.bfloat16)
  b2b_p = jnp.pad(b2b.reshape(_HID, _OUT),
                  ((0, 0), (0, 16 - _OUT))).reshape(1, _HID * 16)
  root2_p = jnp.pad(root2, ((0, 0), (0, 16 - _OUT)))
  bias2_p = jnp.pad(bias2, ((0, 16 - _OUT))).reshape(1, 16)

  b1a_ = b1a.reshape(1, _MLP_H)
  b1b_ = b1b.reshape(1, _IN * _HID)
  b2a_ = b2a.reshape(1, _MLP_H)
  bias1_ = bias1.reshape(1, _HID)

  xs1, cnt = _sc_gather_count(x, src_p, dst_p)
  c0 = cnt[0].reshape(_N_SC, 1)
  c1 = cnt[1].reshape(_N_SC, 1)

  msg1 = _tc_dense(ea_p, W1a_b, b1a_, W1b_b, b1b_, xs1, R16)
  s1 = _sc_scatter_add(msg1, dst_p)
  h = _tc_combine(s1[0], s1[1], c0, c1, x_p, root1, bias1_, relu=True)

  xs2 = _sc_gather(h, src_p)
  msg2 = _tc_dense(ea_p, W2a_b, b2a_, W2b_b, b2b_p, xs2, R16)
  s2 = _sc_scatter_add(msg2, dst_p)
  out = _tc_combine(s2[0], s2[1], c0, c1, h, root2_p, bias2_p, relu=False)

  return out[:_N, :_OUT]


# speedup vs baseline: 1.2983x; 1.0233x over previous
import functools

import jax
import jax.numpy as jnp
from jax import lax
from jax.experimental import pallas as pl
from jax.experimental.pallas import tpu as pltpu
from jax.experimental.pallas import tpu_sc as plsc

_N = 10000
_E = 160000
_IN = 16
_HID = 16
_OUT = 8
_DE = 4
_MLP_H = 25

_NC = 2
_NS = 16
_NW = _NC * _NS
_CH = 128
_NCHUNK = 40
_EPT = _CH * _NCHUNK
_E_PAD = _EPT * _NW
_N_SC = 10240
_RPT = _N_SC // _NS


@functools.cache
def _mesh():
  return plsc.VectorSubcoreMesh(core_axis_name="c", subcore_axis_name="s")


_SC_PARAMS = pltpu.CompilerParams(use_tc_tiling_on_sc=False)


def _sc_gather_count(table, src_idx, dst_idx):
  @functools.partial(
      pl.kernel,
      out_type=[
          jax.ShapeDtypeStruct((_E_PAD, 16), jnp.float32),
          jax.ShapeDtypeStruct((_NC, _N_SC), jnp.float32),
      ],
      mesh=_mesh(),
      compiler_params=_SC_PARAMS,
      scratch_types=[
          pltpu.VMEM((_NCHUNK, _CH), jnp.int32),
          pltpu.VMEM((_NCHUNK, _CH), jnp.int32),
          pltpu.VMEM((_EPT, 16), jnp.float32),
          pltpu.VMEM((_CH,), jnp.float32),
          pltpu.VMEM((_RPT,), jnp.float32),
          pltpu.VMEM_SHARED((_N_SC,), jnp.float32),
          pltpu.SemaphoreType.DMA,
          pltpu.SemaphoreType.DMA,
      ],
  )
  def k(table_hbm, sidx_hbm, didx_hbm, xs_hbm, cnt_hbm,
        sidx_v, didx_v, rows_v, ones_v, cstage_v, cnt_sh, gsem, csem):
    c = lax.axis_index("c")
    s = lax.axis_index("s")
    wid = c * _NS + s
    pltpu.sync_copy(sidx_hbm.at[wid], sidx_v)
    pltpu.sync_copy(didx_hbm.at[wid], didx_v)

    @pl.loop(0, _NCHUNK)
    def _(j):
      pltpu.async_copy(table_hbm.at[sidx_v.at[j]],
                       rows_v.at[pl.ds(j * _CH, _CH)], gsem)

    @pl.loop(0, _CH // 16)
    def _(r):
      ones_v[pl.ds(r * 16, 16)] = jnp.ones((16,), jnp.float32)

    @pl.loop(0, _RPT // 16)
    def _(r):
      cstage_v[pl.ds(r * 16, 16)] = jnp.zeros((16,), jnp.float32)

    pltpu.sync_copy(cstage_v, cnt_sh.at[pl.ds(s * _RPT, _RPT)])
    plsc.subcore_barrier()

    @pl.loop(0, _NCHUNK)
    def _(j):
      pltpu.async_copy(ones_v, cnt_sh.at[didx_v.at[j]], csem, add=True)

    @pl.loop(0, _NCHUNK)
    def _(j):
      pltpu.make_async_copy(ones_v, cnt_sh.at[didx_v.at[j]], csem).wait()

    plsc.subcore_barrier()

    pltpu.sync_copy(cnt_sh.at[pl.ds(s * _RPT, _RPT)], cstage_v)
    pltpu.sync_copy(cstage_v, cnt_hbm.at[c, pl.ds(s * _RPT, _RPT)])

    @pl.loop(0, _NCHUNK)
    def _(j):
      pltpu.make_async_copy(table_hbm.at[sidx_v.at[j]],
                            rows_v.at[pl.ds(j * _CH, _CH)], gsem).wait()

    pltpu.sync_copy(rows_v, xs_hbm.at[pl.ds(wid * _EPT, _EPT)])

  return k(table, src_idx, dst_idx)


def _sc_gather(table, idx):
  @functools.partial(
      pl.kernel,
      out_type=jax.ShapeDtypeStruct((_E_PAD, 16), jnp.float32),
      mesh=_mesh(),
      compiler_params=_SC_PARAMS,
      scratch_types=[
          pltpu.VMEM((_NCHUNK, _CH), jnp.int32),
          pltpu.VMEM((_EPT, 16), jnp.float32),
          pltpu.SemaphoreType.DMA,
      ],
  )
  def k(table_hbm, idx_hbm, out_hbm, idx_v, rows_v, gsem):
    wid = lax.axis_index("c") * _NS + lax.axis_index("s")
    pltpu.sync_copy(idx_hbm.at[wid], idx_v)

    @pl.loop(0, _NCHUNK)
    def _(j):
      pltpu.async_copy(table_hbm.at[idx_v.at[j]],
                       rows_v.at[pl.ds(j * _CH, _CH)], gsem)

    @pl.loop(0, _NCHUNK)
    def _(j):
      pltpu.make_async_copy(table_hbm.at[idx_v.at[j]],
                            rows_v.at[pl.ds(j * _CH, _CH)], gsem).wait()

    pltpu.sync_copy(rows_v, out_hbm.at[pl.ds(wid * _EPT, _EPT)])

  return k(table, idx)


def _sc_scatter_add(msg, idx):
  @functools.partial(
      pl.kernel,
      out_type=jax.ShapeDtypeStruct((_NC, _N_SC, 16), jnp.float32),
      mesh=_mesh(),
      compiler_params=_SC_PARAMS,
      scratch_types=[
          pltpu.VMEM((_NCHUNK, _CH), jnp.int32),
          pltpu.VMEM((_CH, 16), jnp.float32),
          pltpu.VMEM((_CH, 16), jnp.float32),
          pltpu.VMEM((_CH, 16), jnp.float32),
          pltpu.VMEM((_CH, 16), jnp.float32),
          pltpu.VMEM((_RPT, 16), jnp.float32),
          pltpu.VMEM_SHARED((_N_SC, 16), jnp.float32),
          pltpu.SemaphoreType.DMA,
          pltpu.SemaphoreType.DMA,
      ],
  )
  def k(msg_hbm, idx_hbm, out_hbm, idx_v, msg_a, msg_b, msg_c, msg_d,
        stage_v, acc_sh, lsem, ssem):
    c = lax.axis_index("c")
    s = lax.axis_index("s")
    wid = c * _NS + s
    pltpu.sync_copy(idx_hbm.at[wid], idx_v)

    @pl.loop(0, _RPT)
    def _(r):
      stage_v[r, :] = jnp.zeros((16,), jnp.float32)

    pltpu.sync_copy(stage_v, acc_sh.at[pl.ds(s * _RPT, _RPT)])
    plsc.subcore_barrier()

    base = wid * _EPT
    bufs = (msg_a, msg_b, msg_c, msg_d)

    @pl.loop(0, _NCHUNK // 4)
    def _(q):
      j0 = 4 * q

      @pl.when(q > 0)
      def _():
        for k in range(4):
          pltpu.make_async_copy(msg_hbm.at[pl.ds(0, _CH)], bufs[k],
                                ssem).wait()

      for k in range(4):
        pltpu.async_copy(msg_hbm.at[pl.ds(base + (j0 + k) * _CH, _CH)],
                         bufs[k], lsem)
      for k in range(4):
        pltpu.make_async_copy(msg_hbm.at[pl.ds(base + (j0 + k) * _CH, _CH)],
                              bufs[k], lsem).wait()
        pltpu.async_copy(bufs[k], acc_sh.at[idx_v.at[j0 + k]], ssem,
                         add=True)

    @pl.loop(0, 4)
    def _(k):
      pltpu.make_async_copy(msg_hbm.at[pl.ds(0, _CH)], msg_a, ssem).wait()

    plsc.subcore_barrier()

    pltpu.sync_copy(acc_sh.at[pl.ds(s * _RPT, _RPT)], stage_v)
    pltpu.sync_copy(stage_v, out_hbm.at[c, pl.ds(s * _RPT, _RPT)])

  return k(msg, idx)


_EBLK = 8192


def _dense_body(ea_ref, Wa_ref, ba_ref, Wb_ref, bb_ref, xs_ref, R_ref,
                msg_ref):
  dot = functools.partial(jnp.dot, preferred_element_type=jnp.float32)
  bf = jnp.bfloat16
  h = jnp.maximum(dot(ea_ref[...], Wa_ref[...]) + ba_ref[...], 0.0)
  w = dot(h.astype(bf), Wb_ref[...]) + bb_ref[...]
  xrep = dot(xs_ref[...].astype(bf), R_ref[...])
  p = xrep * w
  p = p[:, :128] + p[:, 128:]
  p = p[:, :64] + p[:, 64:]
  p = p[:, :32] + p[:, 32:]
  msg_ref[...] = p[:, :16] + p[:, 16:]


def _tc_dense(ea, Wa, ba, Wb, bb, xs, R):
  grid = _E_PAD // _EBLK
  wdim = Wb.shape[1]
  full = lambda *shape: pl.BlockSpec(shape, lambda i: (0,) * len(shape))
  return pl.pallas_call(
      _dense_body,
      grid=(grid,),
      in_specs=[
          pl.BlockSpec((_EBLK, _DE), lambda i: (i, 0)),
          full(_DE, _MLP_H),
          full(1, _MLP_H),
          full(_MLP_H, wdim),
          full(1, wdim),
          pl.BlockSpec((_EBLK, 16), lambda i: (i, 0)),
          full(16, wdim),
      ],
      out_specs=pl.BlockSpec((_EBLK, 16), lambda i: (i, 0)),
      out_shape=jax.ShapeDtypeStruct((_E_PAD, 16), jnp.float32),
  )(ea, Wa, ba, Wb, bb, xs, R)


def _combine_body(relu, s0_ref, s1_ref, c0_ref, c1_ref, x_ref, root_ref,
                  bias_ref, o_ref):
  cnt = jnp.maximum(c0_ref[...] + c1_ref[...], 1.0)
  out = (s0_ref[...] + s1_ref[...]) / cnt
  out = out + jnp.dot(x_ref[...], root_ref[...],
                      preferred_element_type=jnp.float32) + bias_ref[...]
  if relu:
    out = jnp.maximum(out, 0.0)
  o_ref[...] = out


def _tc_combine(s0, s1, c0, c1, x, root, bias, relu):
  return pl.pallas_call(
      functools.partial(_combine_body, relu),
      out_shape=jax.ShapeDtypeStruct((_N_SC, 16), jnp.float32),
  )(s0, s1, c0, c1, x, root, bias)


def kernel(x, edge_index, edge_attr, W1a, b1a, W1b, b1b, root1, bias1,
           W2a, b2a, W2b, b2b, root2, bias2):
  src = edge_index[0]
  dst = edge_index[1]
  pad = _E_PAD - _E
  src_p = jnp.concatenate([src, jnp.zeros((pad,), jnp.int32)])
  src_p = src_p.reshape(_NW, _NCHUNK, _CH)
  dst_p = jnp.concatenate([dst, jnp.full((pad,), _N, jnp.int32)])
  dst_p = dst_p.reshape(_NW, _NCHUNK, _CH)
  ea_p = jnp.pad(edge_attr, ((0, pad), (0, 0))).astype(jnp.bfloat16)
  x_p = jnp.pad(x, ((0, _N_SC - _N), (0, 0)))

  R16 = jnp.kron(jnp.eye(16, dtype=jnp.bfloat16),
                 jnp.ones((1, 16), jnp.bfloat16))

  W2b_p = jnp.pad(W2b.reshape(_MLP_H, _HID, _OUT),
                  ((0, 0), (0, 0), (0, 16 - _OUT))).reshape(_MLP_H, _HID * 16)
  W1a_b = W1a.astype(jnp.bfloat16)
  W2a_b = W2a.astype(jnp.bfloat16)
  W1b_b = W1b.astype(jnp.bfloat16)
  W2b_b = W2b_p.astype(jnp.bfloat16)
  b2b_p = jnp.pad(b2b.reshape(_HID, _OUT),
                  ((0, 0), (0, 16 - _OUT))).reshape(1, _HID * 16)
  root2_p = jnp.pad(root2, ((0, 0), (0, 16 - _OUT)))
  bias2_p = jnp.pad(bias2, ((0, 16 - _OUT))).reshape(1, 16)

  b1a_ = b1a.reshape(1, _MLP_H)
  b1b_ = b1b.reshape(1, _IN * _HID)
  b2a_ = b2a.reshape(1, _MLP_H)
  bias1_ = bias1.reshape(1, _HID)

  xs1, cnt = _sc_gather_count(x, src_p, dst_p)
  c0 = cnt[0].reshape(_N_SC, 1)
  c1 = cnt[1].reshape(_N_SC, 1)

  msg1 = _tc_dense(ea_p, W1a_b, b1a_, W1b_b, b1b_, xs1, R16)
  s1 = _sc_scatter_add(msg1, dst_p)
  h = _tc_combine(s1[0], s1[1], c0, c1, x_p, root1, bias1_, relu=True)

  xs2 = _sc_gather(h, src_p)
  msg2 = _tc_dense(ea_p, W2a_b, b2a_, W2b_b, b2b_p, xs2, R16)
  s2 = _sc_scatter_add(msg2, dst_p)
  out = _tc_combine(s2[0], s2[1], c0, c1, h, root2_p, bias2_p, relu=False)

  return out[:_N, :_OUT]
